# baseline (speedup 1.0000x reference)
import functools
import jax
import jax.numpy as jnp
from jax import lax
from jax.experimental import pallas as pl
from jax.experimental.pallas import tpu as pltpu
from jax.experimental.pallas import tpu_sc as plsc

_NC, _NS = 2, 16
_NW = _NC * _NS


def kernel(base, source):
    del base
    b, s, d = source.shape
    rows = b * s
    rows_per_w = rows // _NW
    src2d = source.reshape(rows, d)
    mesh = plsc.VectorSubcoreMesh(core_axis_name="c", subcore_axis_name="s")

    @functools.partial(
        pl.kernel,
        mesh=mesh,
        out_type=jax.ShapeDtypeStruct((rows, d), source.dtype),
        scratch_types=[pltpu.SemaphoreType.DMA],
    )
    def _copy(src_hbm, out_hbm, sem):
        wid = lax.axis_index("s") * _NC + lax.axis_index("c")
        base_row = wid * rows_per_w
        pltpu.async_copy(
            src_hbm.at[pl.ds(base_row, rows_per_w)],
            out_hbm.at[pl.ds(base_row, rows_per_w)],
            sem,
        ).wait()

    out = _copy(src2d)
    return out.reshape(b, s, d)


# speedup vs baseline: 36.1730x; 36.1730x over previous
import functools
import jax
import jax.numpy as jnp
from jax import lax
from jax.experimental import pallas as pl
from jax.experimental.pallas import tpu as pltpu
from jax.experimental.pallas import tpu_sc as plsc

_NC, _NS = 2, 16
_NW = _NC * _NS
_CHUNK_ROWS = 32


def kernel(base, source):
    del base
    b, s, d = source.shape
    rows = b * s
    rows_per_w = rows // _NW
    nchunks = rows_per_w // _CHUNK_ROWS
    src2d = source.reshape(rows, d)
    mesh = plsc.VectorSubcoreMesh(core_axis_name="c", subcore_axis_name="s")

    @functools.partial(
        pl.kernel,
        mesh=mesh,
        out_type=jax.ShapeDtypeStruct((rows, d), source.dtype),
        scratch_types=[
            pltpu.VMEM((2, _CHUNK_ROWS, d), jnp.float32),
            pltpu.SemaphoreType.DMA((2,)),
            pltpu.SemaphoreType.DMA((2,)),
        ],
    )
    def _copy(src_hbm, out_hbm, buf, in_sems, out_sems):
        wid = lax.axis_index("s") * _NC + lax.axis_index("c")
        base_row = wid * rows_per_w

        def _in(g, bslot):
            return pltpu.make_async_copy(
                src_hbm.at[pl.ds(base_row + g * _CHUNK_ROWS, _CHUNK_ROWS)],
                buf.at[bslot],
                in_sems.at[bslot],
            )

        def _out(g, bslot):
            return pltpu.make_async_copy(
                buf.at[bslot],
                out_hbm.at[pl.ds(base_row + g * _CHUNK_ROWS, _CHUNK_ROWS)],
                out_sems.at[bslot],
            )

        _in(0, 0).start()
        _in(1, 1).start()
        for g in range(nchunks):
            bslot = g % 2
            _in(g, bslot).wait()
            _out(g, bslot).start()
            if g + 2 < nchunks:
                _out(g, bslot).wait()
                _in(g + 2, bslot).start()
        _out(nchunks - 2, (nchunks - 2) % 2).wait()
        _out(nchunks - 1, (nchunks - 1) % 2).wait()

    out = _copy(src2d)
    return out.reshape(b, s, d)
